# baseline (device time: 1061142 ns/iter reference)
import jax
import jax.numpy as jnp
from jax import lax
from jax.experimental import pallas as pl
from jax.experimental.pallas import tpu as pltpu

CHUNK = 128


def kernel(x, W):
    T, D = x.shape
    _, V = W.shape
    n_chunks = T // CHUNK
    K = n_chunks // 2

    logits = jnp.dot(x, W, precision=lax.Precision.DEFAULT)

    def body(logits_ref, out_ref, loc_ref, xr_ref, yr_ref,
             xsend_sems, xrecv_sems, ysend_sems, yrecv_sems,
             xcredit, ycredit):
        j = pl.program_id(0)
        my_x = lax.axis_index("x")
        my_y = lax.axis_index("y")
        xn = (1 - my_x, my_y)
        yn = (my_x, 1 - my_y)
        k = j // 2
        s2 = lax.rem(k, 2)
        s4 = lax.rem(k, 4)

        def xdesc(slot4, slot2):
            return pltpu.make_async_remote_copy(
                src_ref=loc_ref.at[slot2],
                dst_ref=xr_ref.at[slot4],
                send_sem=xsend_sems.at[slot2],
                recv_sem=xrecv_sems.at[slot4],
                device_id=xn,
                device_id_type=pl.DeviceIdType.MESH,
            )

        def ydesc(slot4):
            return pltpu.make_async_remote_copy(
                src_ref=xr_ref.at[slot4],
                dst_ref=yr_ref.at[slot4],
                send_sem=ysend_sems.at[slot4],
                recv_sem=yrecv_sems.at[slot4],
                device_id=yn,
                device_id_type=pl.DeviceIdType.MESH,
            )

        def compute(lrem_bf16):
            eloc = jnp.exp(logits_ref[...])
            erem = jnp.exp(lrem_bf16)
            denom = (
                jnp.sum(eloc, axis=-1, keepdims=True)
                + jnp.sum(erem, axis=-1, keepdims=True, dtype=jnp.float32)
            )
            r = 1.0 / denom
            out_ref[:, pl.ds(my_x * V, V)] = (eloc * r).astype(jnp.bfloat16)
            out_ref[:, pl.ds((1 - my_x) * V, V)] = (
                erem.astype(jnp.float32) * r
            ).astype(jnp.bfloat16)

        @pl.when(j == 0)
        def _():
            bar = pltpu.get_barrier_semaphore()
            for nbr in (xn, yn):
                pl.semaphore_signal(
                    bar, inc=1, device_id=nbr,
                    device_id_type=pl.DeviceIdType.MESH,
                )
            pl.semaphore_wait(bar, 2)

        in_range = j < n_chunks
        is_a = jnp.logical_and(in_range, lax.rem(j, 2) == my_y)
        is_b = jnp.logical_and(in_range, lax.rem(j, 2) != my_y)

        @pl.when(is_a)
        def _():
            @pl.when(k >= 2)
            def _():
                xdesc(s4, s2).wait_send()
            @pl.when(k >= 4)
            def _():
                pl.semaphore_wait(xcredit, 1)

            loc_ref[s2] = logits_ref[...].astype(jnp.bfloat16)
            xdesc(s4, s2).start()
            xdesc(s4, s2).wait_recv()

            @pl.when(k >= 2)
            def _():
                ydesc(lax.rem(k - 2, 4)).wait_send()
                @pl.when(k <= 5)
                def _():
                    pl.semaphore_signal(
                        xcredit, inc=1, device_id=xn,
                        device_id_type=pl.DeviceIdType.MESH,
                    )
            @pl.when(k >= 4)
            def _():
                pl.semaphore_wait(ycredit, 1)
            ydesc(s4).start()

            compute(xr_ref[s4])

        @pl.when(is_b)
        def _():
            ydesc(s4).wait_recv()
            compute(yr_ref[s4])
            @pl.when(k <= 3)
            def _():
                pl.semaphore_signal(
                    ycredit, inc=1, device_id=yn,
                    device_id_type=pl.DeviceIdType.MESH,
                )

        @pl.when(j == n_chunks)
        def _():
            xdesc(0, 0).wait_send()
            xdesc(0, 1).wait_send()
            ydesc(2).wait_send()
            ydesc(3).wait_send()

    def chunk_index(j):
        jc = jnp.minimum(j, n_chunks - 1)
        return ((jc // 2) + K * (jc % 2), 0)

    return pl.pallas_call(
        body,
        grid=(n_chunks + 1,),
        in_specs=[pl.BlockSpec((CHUNK, V), chunk_index)],
        out_specs=pl.BlockSpec((CHUNK, 2 * V), chunk_index),
        out_shape=jax.ShapeDtypeStruct((T, 2 * V), jnp.bfloat16),
        scratch_shapes=[
            pltpu.VMEM((2, CHUNK, V), jnp.bfloat16),
            pltpu.VMEM((4, CHUNK, V), jnp.bfloat16),
            pltpu.VMEM((4, CHUNK, V), jnp.bfloat16),
            pltpu.SemaphoreType.DMA((2,)),
            pltpu.SemaphoreType.DMA((4,)),
            pltpu.SemaphoreType.DMA((4,)),
            pltpu.SemaphoreType.DMA((4,)),
            pltpu.SemaphoreType.REGULAR,
            pltpu.SemaphoreType.REGULAR,
        ],
        compiler_params=pltpu.CompilerParams(
            collective_id=0,
            vmem_limit_bytes=56 * 1024 * 1024,
        ),
    )(logits)


# device time: 538688 ns/iter; 1.9699x vs baseline; 1.9699x over previous
import jax
import jax.numpy as jnp
from jax import lax
from jax.experimental import pallas as pl
from jax.experimental.pallas import tpu as pltpu

CHUNK = 128


def kernel(x, W):
    T, D = x.shape
    _, V = W.shape
    n_chunks = T // CHUNK
    K = n_chunks // 2

    logits = jnp.dot(x, W, precision=lax.Precision.DEFAULT).astype(
        jnp.bfloat16
    )

    def body(lg0_ref, lg1_ref, outa_ref, outb_ref,
             loc_ref, othr_ref, xr_ref, yr_ref,
             xsend_sems, xrecv_sems, ysend_sems, yrecv_sems,
             xcredit, ycredit):
        q = pl.program_id(0)
        my_x = lax.axis_index("x")
        my_y = lax.axis_index("y")
        xn = (1 - my_x, my_y)
        yn = (my_x, 1 - my_y)
        s2 = lax.rem(q, 2)
        s4 = lax.rem(q, 4)
        p2 = lax.rem(q + 1, 2)
        p4 = lax.rem(q + 3, 4)

        def xdesc(slot4, slot2):
            return pltpu.make_async_remote_copy(
                src_ref=loc_ref.at[slot2],
                dst_ref=xr_ref.at[slot4],
                send_sem=xsend_sems.at[slot2],
                recv_sem=xrecv_sems.at[slot4],
                device_id=xn,
                device_id_type=pl.DeviceIdType.MESH,
            )

        def ydesc(slot4):
            return pltpu.make_async_remote_copy(
                src_ref=xr_ref.at[slot4],
                dst_ref=yr_ref.at[slot4],
                send_sem=ysend_sems.at[slot4],
                recv_sem=yrecv_sems.at[slot4],
                device_id=yn,
                device_id_type=pl.DeviceIdType.MESH,
            )

        def softmax_store(out_ref, lloc_bf16, lrem_bf16):
            eloc = jnp.exp(lloc_bf16)
            erem = jnp.exp(lrem_bf16)
            denom = (
                jnp.sum(eloc, axis=-1, keepdims=True, dtype=jnp.float32)
                + jnp.sum(erem, axis=-1, keepdims=True, dtype=jnp.float32)
            )
            r = 1.0 / denom
            out_ref[:, pl.ds(my_x * V, V)] = (
                eloc.astype(jnp.float32) * r
            ).astype(jnp.bfloat16)
            out_ref[:, pl.ds((1 - my_x) * V, V)] = (
                erem.astype(jnp.float32) * r
            ).astype(jnp.bfloat16)

        @pl.when(q == 0)
        def _():
            bar = pltpu.get_barrier_semaphore()
            for nbr in (xn, yn):
                pl.semaphore_signal(
                    bar, inc=1, device_id=nbr,
                    device_id_type=pl.DeviceIdType.MESH,
                )
            pl.semaphore_wait(bar, 2)

        @pl.when(q < K)
        def _():
            @pl.when(q >= 2)
            def _():
                xdesc(s4, s2).wait_send()
            @pl.when(q >= 4)
            def _():
                pl.semaphore_wait(xcredit, 1)

            @pl.when(my_y == 0)
            def _():
                loc_ref[s2] = lg0_ref[...]
                othr_ref[s2] = lg1_ref[...]
            @pl.when(my_y == 1)
            def _():
                loc_ref[s2] = lg1_ref[...]
                othr_ref[s2] = lg0_ref[...]

            xdesc(s4, s2).start()
            xdesc(s4, s2).wait_recv()

            @pl.when(q >= 2)
            def _():
                ydesc(lax.rem(q + 2, 4)).wait_send()
                @pl.when(q <= 5)
                def _():
                    pl.semaphore_signal(
                        xcredit, inc=1, device_id=xn,
                        device_id_type=pl.DeviceIdType.MESH,
                    )
            @pl.when(q >= 4)
            def _():
                pl.semaphore_wait(ycredit, 1)
            ydesc(s4).start()

            softmax_store(outa_ref, loc_ref[s2], xr_ref[s4])

        @pl.when(jnp.logical_and(q >= 1, q <= K))
        def _():
            ydesc(p4).wait_recv()
            softmax_store(outb_ref, othr_ref[p2], yr_ref[p4])
            @pl.when(q <= 4)
            def _():
                pl.semaphore_signal(
                    ycredit, inc=1, device_id=yn,
                    device_id_type=pl.DeviceIdType.MESH,
                )

        @pl.when(q == K + 1)
        def _():
            xdesc(0, 0).wait_send()
            xdesc(0, 1).wait_send()
            ydesc(2).wait_send()
            ydesc(3).wait_send()

    qa = lambda q: (jnp.minimum(q, K - 1), 0)
    qb = lambda q: (jnp.clip(q - 1, 0, K - 1), 0)

    outa, outb = pl.pallas_call(
        body,
        grid=(K + 2,),
        in_specs=[
            pl.BlockSpec((CHUNK, V), qa),
            pl.BlockSpec((CHUNK, V), lambda q: (K + jnp.minimum(q, K - 1), 0)),
        ],
        out_specs=[
            pl.BlockSpec((CHUNK, 2 * V), qa),
            pl.BlockSpec((CHUNK, 2 * V), qb),
        ],
        out_shape=[
            jax.ShapeDtypeStruct((K * CHUNK, 2 * V), jnp.bfloat16),
            jax.ShapeDtypeStruct((K * CHUNK, 2 * V), jnp.bfloat16),
        ],
        scratch_shapes=[
            pltpu.VMEM((2, CHUNK, V), jnp.bfloat16),
            pltpu.VMEM((2, CHUNK, V), jnp.bfloat16),
            pltpu.VMEM((4, CHUNK, V), jnp.bfloat16),
            pltpu.VMEM((4, CHUNK, V), jnp.bfloat16),
            pltpu.SemaphoreType.DMA((2,)),
            pltpu.SemaphoreType.DMA((4,)),
            pltpu.SemaphoreType.DMA((4,)),
            pltpu.SemaphoreType.DMA((4,)),
            pltpu.SemaphoreType.REGULAR,
            pltpu.SemaphoreType.REGULAR,
        ],
        compiler_params=pltpu.CompilerParams(
            collective_id=0,
            vmem_limit_bytes=60 * 1024 * 1024,
        ),
    )(logits, logits)

    my_y = lax.axis_index("y")
    return lax.cond(
        my_y == 0,
        lambda: jnp.concatenate([outa, outb], axis=0),
        lambda: jnp.concatenate([outb, outa], axis=0),
    )
